# baseline (speedup 1.0000x reference)
import functools

import jax
import jax.numpy as jnp
from jax import lax
from jax.experimental import pallas as pl
from jax.experimental.pallas import tpu as pltpu
from jax.experimental.pallas import tpu_sc as plsc

D = 64
T = 200
NC = 2
NS = 16
NW = NC * NS
IDXW = 100
CS = 2
CHUNK = CS * T
NSUB = CHUNK // IDXW


def _body(x2d, tok, pos, out, idx_v, buf_v, pos_v, sem):
    c = lax.axis_index("c")
    s = lax.axis_index("s")
    wid = s * NC + c

    n_rows_total = out.shape[0]
    rows_per_w = n_rows_total // NW
    chunks_per_w = rows_per_w // CHUNK
    idx_rows_per_w = rows_per_w // IDXW

    pltpu.sync_copy(pos, pos_v)

    def chunk_body(g, carry):
        idx_r0 = wid * idx_rows_per_w + g * NSUB
        pltpu.sync_copy(x2d.at[pl.ds(idx_r0, NSUB)], idx_v)
        cps = []
        for j in range(NSUB):
            cps.append(
                pltpu.async_copy(
                    tok.at[idx_v.at[j]],
                    buf_v.at[pl.ds(j * IDXW, IDXW)],
                    sem,
                )
            )
        for cp in cps:
            cp.wait()

        def t_body(t, c2):
            for k in range(D // 16):
                p = pos_v[t, pl.ds(k * 16, 16)]
                for sq in range(CS):
                    plsc.addupdate(buf_v.at[sq * T + t, pl.ds(k * 16, 16)], p)
            return c2

        lax.fori_loop(0, T, t_body, 0)

        row0 = wid * rows_per_w + g * CHUNK
        pltpu.sync_copy(buf_v, out.at[pl.ds(row0, CHUNK)])
        return carry

    lax.fori_loop(0, chunks_per_w, chunk_body, 0)


@functools.partial(jax.jit, static_argnums=(3,))
def _embed(x2d, tok_emb, pos2d, n_rows):
    run = pl.kernel(
        _body,
        out_type=jax.ShapeDtypeStruct((n_rows, D), jnp.float32),
        mesh=plsc.VectorSubcoreMesh(core_axis_name="c", subcore_axis_name="s"),
        scratch_types=[
            pltpu.VMEM((NSUB, IDXW), jnp.int32),
            pltpu.VMEM((CHUNK, D), jnp.float32),
            pltpu.VMEM((T, D), jnp.float32),
            pltpu.SemaphoreType.DMA,
        ],
        compiler_params=pltpu.CompilerParams(use_tc_tiling_on_sc=False),
    )
    return run(x2d, tok_emb, pos2d)


def kernel(x, tok_emb, pos_emb):
    b, t = x.shape
    n_rows = b * t
    x2d = x.astype(jnp.int32).reshape(n_rows // IDXW, IDXW)
    pos2d = pos_emb[0, :t, :]
    out = _embed(x2d, tok_emb, pos2d, n_rows)
    return out.reshape(b, t, D)


# speedup vs baseline: 1.1285x; 1.1285x over previous
import functools

import jax
import jax.numpy as jnp
from jax import lax
from jax.experimental import pallas as pl
from jax.experimental.pallas import tpu as pltpu
from jax.experimental.pallas import tpu_sc as plsc

D = 64
T = 200
NC = 2
NS = 16
NW = NC * NS
IDXW = 100
CHUNK = T
NSUB = CHUNK // IDXW
NBUF = 4
AHEAD = 3


def _body(x2d, tok, pos, out, idx_v, pos_v, bufs, gsems, ssems, isems):
    c = lax.axis_index("c")
    s = lax.axis_index("s")
    wid = s * NC + c

    n_rows_total = out.shape[0]
    rows_per_w = n_rows_total // NW
    n_chunks = rows_per_w // CHUNK
    idx_rows_per_w = rows_per_w // IDXW

    def idx_rows(g):
        return wid * idx_rows_per_w + g * NSUB

    def out_row0(g):
        return wid * rows_per_w + g * CHUNK

    def fire_gathers(g, b):
        cps = []
        for j in range(NSUB):
            cps.append(
                pltpu.async_copy(
                    tok.at[idx_v.at[b, j]],
                    bufs[b].at[pl.ds(j * IDXW, IDXW)],
                    gsems[b],
                )
            )
        return cps

    def wait_gathers(b):
        for j in range(NSUB):
            pltpu.make_async_copy(
                tok.at[idx_v.at[b, j]],
                bufs[b].at[pl.ds(j * IDXW, IDXW)],
                gsems[b],
            ).wait()

    def fire_idx(g, b):
        pltpu.async_copy(x2d.at[pl.ds(idx_rows(g), NSUB)], idx_v.at[b], isems[b])

    def wait_idx(b):
        pltpu.make_async_copy(
            x2d.at[pl.ds(0, NSUB)], idx_v.at[b], isems[b]
        ).wait()

    def fire_scatter(g, b):
        pltpu.async_copy(bufs[b], out.at[pl.ds(out_row0(g), CHUNK)], ssems[b])

    def wait_scatter(b):
        pltpu.make_async_copy(
            bufs[b], out.at[pl.ds(0, CHUNK)], ssems[b]
        ).wait()

    pltpu.sync_copy(pos, pos_v)

    for b in range(AHEAD):
        pltpu.sync_copy(x2d.at[pl.ds(idx_rows(b), NSUB)], idx_v.at[b])
        fire_gathers(b, b)
    fire_idx(AHEAD, AHEAD)

    def outer(g2, carry):
        for b in range(NBUF):
            g = g2 * NBUF + b
            buf = bufs[b]

            wait_gathers(b)

            @pl.when(g + NBUF < n_chunks)
            def _():
                fire_idx(g + NBUF, b)

            @plsc.parallel_loop(0, T, unroll=4)
            def _(t):
                for k in range(D // 16):
                    sl = pl.ds(k * 16, 16)
                    plsc.addupdate(buf.at[t, sl], pos_v[t, sl])

            fire_scatter(g, b)

            jn = (b + AHEAD) % NBUF

            @pl.when(g >= 1)
            def _():
                wait_scatter(jn)

            @pl.when(g + AHEAD < n_chunks)
            def _():
                wait_idx(jn)
                fire_gathers(g + AHEAD, jn)

        return carry

    lax.fori_loop(0, n_chunks // NBUF, outer, 0)

    wait_scatter(NBUF - 1)


def _entry(x2d, tok, pos, out, idx_v, pos_v,
           b0, b1, b2, b3, g0, g1, g2, g3, s0, s1, s2, s3, i0, i1, i2, i3):
    _body(x2d, tok, pos, out, idx_v, pos_v,
          [b0, b1, b2, b3], [g0, g1, g2, g3], [s0, s1, s2, s3],
          [i0, i1, i2, i3])


@functools.partial(jax.jit, static_argnums=(3,))
def _embed(x2d, tok_emb, pos2d, n_rows):
    run = pl.kernel(
        _entry,
        out_type=jax.ShapeDtypeStruct((n_rows, D), jnp.float32),
        mesh=plsc.VectorSubcoreMesh(core_axis_name="c", subcore_axis_name="s"),
        scratch_types=(
            [pltpu.VMEM((NBUF, NSUB, IDXW), jnp.int32),
             pltpu.VMEM((T, D), jnp.float32)]
            + [pltpu.VMEM((CHUNK, D), jnp.float32) for _ in range(NBUF)]
            + [pltpu.SemaphoreType.DMA for _ in range(3 * NBUF)]
        ),
        compiler_params=pltpu.CompilerParams(use_tc_tiling_on_sc=False),
    )
    return run(x2d, tok_emb, pos2d)


def kernel(x, tok_emb, pos_emb):
    b, t = x.shape
    n_rows = b * t
    x2d = x.astype(jnp.int32).reshape(n_rows // IDXW, IDXW)
    pos2d = pos_emb[0, :t, :]
    out = _embed(x2d, tok_emb, pos2d, n_rows)
    return out.reshape(b, t, D)
